# Initial kernel scaffold; baseline (speedup 1.0000x reference)
import functools

import jax
import jax.numpy as jnp
from jax import lax
from jax.experimental import pallas as pl
from jax.experimental.pallas import tpu as pltpu
from jax.experimental.pallas import tpu_sc as plsc

N = 10000
E = 320000
D = 128
H2 = 128
C = 64

NC, NS = 2, 16
NW = NC * NS
CW = 128
CHUNKS = -(-E // (NW * CW))
EPW = CHUNKS * CW
EPAD = EPW * NW
RPT = N // NS

_MESH = dict(core_axis_name="c", subcore_axis_name="s", num_cores=NC,
             num_subcores=NS)


def _wid():
    return lax.axis_index("c") * NS + lax.axis_index("s")


def _sc_degree(dst_p, zeros16, ones16):
    @functools.partial(
        pl.kernel,
        out_type=jax.ShapeDtypeStruct((NC, N, 16), jnp.float32),
        mesh=plsc.VectorSubcoreMesh(**_MESH),
        scratch_types=[
            pltpu.VMEM((CHUNKS, CW), jnp.int32),
            pltpu.VMEM((CW, 16), jnp.float32),
            pltpu.VMEM_SHARED((N + 1, 16), jnp.float32),
        ],
    )
    def k(dst_hbm, zeros_hbm, ones_hbm, out_hbm, didx, ones_v, acc):
        cid = lax.axis_index("c")
        sid = lax.axis_index("s")
        w = _wid()
        pltpu.sync_copy(dst_hbm.at[w], didx)
        pltpu.sync_copy(ones_hbm, ones_v)
        pltpu.sync_copy(zeros_hbm.at[pl.ds(sid * RPT, RPT)],
                        acc.at[pl.ds(sid * RPT, RPT)])

        @pl.when(sid == NS - 1)
        def _():
            pltpu.sync_copy(zeros_hbm.at[pl.ds(N, 1)], acc.at[pl.ds(N, 1)])

        plsc.subcore_barrier()

        def chunk(j, carry):
            pltpu.sync_copy(ones_v, acc.at[didx.at[j]], add=True)
            return carry

        lax.fori_loop(0, CHUNKS, chunk, 0)
        plsc.subcore_barrier()
        pltpu.sync_copy(acc.at[pl.ds(sid * RPT, RPT)],
                        out_hbm.at[cid, pl.ds(sid * RPT, RPT)])

    return k(dst_p, zeros16, ones16)


def _make_conv(Dw):
    @functools.partial(
        pl.kernel,
        out_type=jax.ShapeDtypeStruct((NC, N, Dw), jnp.float32),
        mesh=plsc.VectorSubcoreMesh(**_MESH),
        scratch_types=[
            pltpu.VMEM((CHUNKS, CW), jnp.int32),
            pltpu.VMEM((CHUNKS, CW), jnp.int32),
            pltpu.VMEM((CW, Dw), jnp.float32),
            pltpu.VMEM((CW, Dw), jnp.float32),
            pltpu.VMEM_SHARED((N + 1, Dw), jnp.float32),
            pltpu.SemaphoreType.DMA,
            pltpu.SemaphoreType.DMA,
        ],
    )
    def k(src_hbm, dst_hbm, xw_hbm, zeros_hbm, out_hbm,
          sidx, didx, buf0, buf1, acc, sem0, sem1):
        cid = lax.axis_index("c")
        sid = lax.axis_index("s")
        w = _wid()
        pltpu.sync_copy(src_hbm.at[w], sidx)
        pltpu.sync_copy(dst_hbm.at[w], didx)
        pltpu.sync_copy(zeros_hbm.at[pl.ds(sid * RPT, RPT)],
                        acc.at[pl.ds(sid * RPT, RPT)])

        @pl.when(sid == NS - 1)
        def _():
            pltpu.sync_copy(zeros_hbm.at[pl.ds(N, 1)], acc.at[pl.ds(N, 1)])

        plsc.subcore_barrier()

        pltpu.async_copy(xw_hbm.at[sidx.at[0]], buf0, sem0)

        def pair(p, carry):
            j = 2 * p
            pltpu.async_copy(xw_hbm.at[sidx.at[j + 1]], buf1, sem1)
            pltpu.make_async_copy(xw_hbm.at[sidx.at[j]], buf0, sem0).wait()
            pltpu.sync_copy(buf0, acc.at[didx.at[j]], add=True)
            pltpu.async_copy(xw_hbm.at[sidx.at[j + 2]], buf0, sem0)
            pltpu.make_async_copy(xw_hbm.at[sidx.at[j + 1]], buf1, sem1).wait()
            pltpu.sync_copy(buf1, acc.at[didx.at[j + 1]], add=True)
            return carry

        lax.fori_loop(0, (CHUNKS - 1) // 2, pair, 0)
        pltpu.make_async_copy(xw_hbm.at[sidx.at[CHUNKS - 1]], buf0, sem0).wait()
        pltpu.sync_copy(buf0, acc.at[didx.at[CHUNKS - 1]], add=True)

        plsc.subcore_barrier()
        pltpu.sync_copy(acc.at[pl.ds(sid * RPT, RPT)],
                        out_hbm.at[cid, pl.ds(sid * RPT, RPT)])

    return k


_conv128 = _make_conv(D)
_conv64 = _make_conv(C)


def _sc_decode(la_p, lb_p, z):
    @functools.partial(
        pl.kernel,
        out_type=jax.ShapeDtypeStruct((NW, CHUNKS, CW), jnp.float32),
        mesh=plsc.VectorSubcoreMesh(**_MESH),
        scratch_types=[
            pltpu.VMEM((CHUNKS, CW), jnp.int32),
            pltpu.VMEM((CHUNKS, CW), jnp.int32),
            pltpu.VMEM((CW, C), jnp.float32),
            pltpu.VMEM((CW, C), jnp.float32),
            pltpu.VMEM((CW, C), jnp.float32),
            pltpu.VMEM((CW, C), jnp.float32),
            pltpu.VMEM((CHUNKS, CW), jnp.float32),
            pltpu.SemaphoreType.DMA,
            pltpu.SemaphoreType.DMA,
        ],
    )
    def k(la_hbm, lb_hbm, z_hbm, out_hbm,
          aidx, bidx, a0, b0, a1, b1, outv, semA, semB):
        w = _wid()
        pltpu.sync_copy(la_hbm.at[w], aidx)
        pltpu.sync_copy(lb_hbm.at[w], bidx)

        def fire(j, ab, bb, sem):
            pltpu.async_copy(z_hbm.at[aidx.at[j]], ab, sem)
            pltpu.async_copy(z_hbm.at[bidx.at[j]], bb, sem)

        def drain(j, ab, bb, sem):
            pltpu.make_async_copy(z_hbm.at[aidx.at[j]], ab, sem).wait()
            pltpu.make_async_copy(z_hbm.at[bidx.at[j]], bb, sem).wait()

        def compute(j, ab, bb):
            def edge(e, carry):
                p0 = ab[e, pl.ds(0, 16)] * bb[e, pl.ds(0, 16)]
                p1 = ab[e, pl.ds(16, 16)] * bb[e, pl.ds(16, 16)]
                p2 = ab[e, pl.ds(32, 16)] * bb[e, pl.ds(32, 16)]
                p3 = ab[e, pl.ds(48, 16)] * bb[e, pl.ds(48, 16)]
                s = (p0 + p1) + (p2 + p3)
                outv[j, e] = jnp.sum(s)
                return carry

            lax.fori_loop(0, CW, edge, 0)

        fire(0, a0, b0, semA)

        def pair(p, carry):
            j = 2 * p
            fire(j + 1, a1, b1, semB)
            drain(j, a0, b0, semA)
            compute(j, a0, b0)
            fire(j + 2, a0, b0, semA)
            drain(j + 1, a1, b1, semB)
            compute(j + 1, a1, b1)
            return carry

        lax.fori_loop(0, (CHUNKS - 1) // 2, pair, 0)
        drain(CHUNKS - 1, a0, b0, semA)
        compute(CHUNKS - 1, a0, b0)

        pltpu.sync_copy(outv, out_hbm.at[w])

    return k(la_p, lb_p, z)


_RB = 1000


def _dinv_of(degp):
    deg = degp[0, :, 0:1] + degp[1, :, 0:1] + 1.0
    return lax.rsqrt(deg)


def _tc1_body(x_ref, wih_ref, bias_ref, w1_ref, degp_ref, xw1p_ref):
    x = x_ref[...]
    g = lax.dot_general(x, wih_ref[...], (((1,), (1,)), ((), ())),
                        preferred_element_type=jnp.float32)
    g = g + bias_ref[...]
    i = g[:, 0:D]
    gg = g[:, 2 * D:3 * D]
    o = g[:, 3 * D:4 * D]
    c = jax.nn.sigmoid(i) * jnp.tanh(gg)
    h = jax.nn.sigmoid(o) * jnp.tanh(c)
    dinv = _dinv_of(degp_ref[...])
    xw1p_ref[...] = jnp.dot(h, w1_ref[...],
                            preferred_element_type=jnp.float32) * dinv


def _tc1(x, W_ih, bias, W1, degp):
    grid = N // _RB
    return pl.pallas_call(
        _tc1_body,
        grid=(grid,),
        in_specs=[
            pl.BlockSpec((_RB, D), lambda b: (b, 0)),
            pl.BlockSpec((4 * D, D), lambda b: (0, 0)),
            pl.BlockSpec((1, 4 * D), lambda b: (0, 0)),
            pl.BlockSpec((D, H2), lambda b: (0, 0)),
            pl.BlockSpec((NC, _RB, 16), lambda b: (0, b, 0)),
        ],
        out_specs=pl.BlockSpec((_RB, H2), lambda b: (b, 0)),
        out_shape=jax.ShapeDtypeStruct((N, H2), jnp.float32),
    )(x, W_ih, bias, W1, degp)


def _tc2_body(accp_ref, xw1p_ref, degp_ref, b1_ref, w2_ref, xw2p_ref):
    dinv = _dinv_of(degp_ref[...])
    s = accp_ref[0] + accp_ref[1] + xw1p_ref[...]
    h1 = jnp.maximum(dinv * s + b1_ref[...], 0.0)
    xw2p_ref[...] = jnp.dot(h1, w2_ref[...],
                            preferred_element_type=jnp.float32) * dinv


def _tc2(acc1, xw1p, degp, b1r, W2):
    grid = N // _RB
    return pl.pallas_call(
        _tc2_body,
        grid=(grid,),
        in_specs=[
            pl.BlockSpec((NC, _RB, H2), lambda b: (0, b, 0)),
            pl.BlockSpec((_RB, H2), lambda b: (b, 0)),
            pl.BlockSpec((NC, _RB, 16), lambda b: (0, b, 0)),
            pl.BlockSpec((1, H2), lambda b: (0, 0)),
            pl.BlockSpec((H2, C), lambda b: (0, 0)),
        ],
        out_specs=pl.BlockSpec((_RB, C), lambda b: (b, 0)),
        out_shape=jax.ShapeDtypeStruct((N, C), jnp.float32),
    )(acc1, xw1p, degp, b1r, W2)


def _tc3_body(accp_ref, xw2p_ref, degp_ref, b2_ref, z_ref):
    dinv = _dinv_of(degp_ref[...])
    z_ref[...] = dinv * (accp_ref[0] + accp_ref[1] + xw2p_ref[...]) \
        + b2_ref[...]


def _tc3(acc2, xw2p, degp, b2r):
    grid = N // _RB
    return pl.pallas_call(
        _tc3_body,
        grid=(grid,),
        in_specs=[
            pl.BlockSpec((NC, _RB, C), lambda b: (0, b, 0)),
            pl.BlockSpec((_RB, C), lambda b: (b, 0)),
            pl.BlockSpec((NC, _RB, 16), lambda b: (0, b, 0)),
            pl.BlockSpec((1, C), lambda b: (0, 0)),
        ],
        out_specs=pl.BlockSpec((_RB, C), lambda b: (b, 0)),
        out_shape=jax.ShapeDtypeStruct((N, C), jnp.float32),
    )(acc2, xw2p, degp, b2r)


def kernel(x, edge_index, edge_label_index, W_ih, W_hh, b_ih, b_hh,
           W1, b1, W2, b2):
    pad = EPAD - E
    zpad = jnp.zeros((pad,), jnp.int32)
    src_p = jnp.concatenate([edge_index[0], zpad]).reshape(NW, CHUNKS, CW)
    dst_p = jnp.concatenate(
        [edge_index[1], jnp.full((pad,), N, jnp.int32)]
    ).reshape(NW, CHUNKS, CW)
    la_p = jnp.concatenate([edge_label_index[0], zpad]).reshape(NW, CHUNKS, CW)
    lb_p = jnp.concatenate([edge_label_index[1], zpad]).reshape(NW, CHUNKS, CW)

    zeros128 = jnp.zeros((N + 1, D), jnp.float32)
    zeros64 = jnp.zeros((N + 1, C), jnp.float32)
    zeros16 = jnp.zeros((N + 1, 16), jnp.float32)
    ones16 = jnp.ones((CW, 16), jnp.float32)
    bias = (b_ih + b_hh).reshape(1, 4 * D)
    b1r = b1.reshape(1, H2)
    b2r = b2.reshape(1, C)

    degp = _sc_degree(dst_p, zeros16, ones16)
    xw1p = _tc1(x, W_ih, bias, W1, degp)
    acc1 = _conv128(src_p, dst_p, xw1p, zeros128)
    xw2p = _tc2(acc1, xw1p, degp, b1r, W2)
    acc2 = _conv64(src_p, dst_p, xw2p, zeros64)
    z = _tc3(acc2, xw2p, degp, b2r)
    outp = _sc_decode(la_p, lb_p, z)
    return outp.reshape(-1)[:E]


# speedup vs baseline: 7.0869x; 7.0869x over previous
import functools

import jax
import jax.numpy as jnp
from jax import lax
from jax.experimental import pallas as pl
from jax.experimental.pallas import tpu as pltpu
from jax.experimental.pallas import tpu_sc as plsc

N = 10000
E = 320000
D = 128
H2 = 128
C = 64

NC, NS = 2, 16
NW = NC * NS
CW = 128
CHUNKS = -(-E // (NW * CW))
EPW = CHUNKS * CW
EPAD = EPW * NW
ZROW = 632
ZLAST = N - ZROW * (NS - 1)
AROWS = N + 8

_MESH = dict(core_axis_name="c", subcore_axis_name="s", num_cores=NC,
             num_subcores=NS)


def _wid():
    return lax.axis_index("c") * NS + lax.axis_index("s")


def _init_acc(sid, zeros_hbm, acc):
    @pl.when(sid < NS - 1)
    def _():
        pltpu.sync_copy(zeros_hbm.at[pl.ds(sid * ZROW, ZROW)],
                        acc.at[pl.ds(sid * ZROW, ZROW)])

    @pl.when(sid == NS - 1)
    def _():
        pltpu.sync_copy(zeros_hbm.at[pl.ds((NS - 1) * ZROW, ZLAST)],
                        acc.at[pl.ds((NS - 1) * ZROW, ZLAST)])


def _read_acc(cid, sid, acc, out_hbm):
    @pl.when(sid < NS - 1)
    def _():
        pltpu.sync_copy(acc.at[pl.ds(sid * ZROW, ZROW)],
                        out_hbm.at[cid, pl.ds(sid * ZROW, ZROW)])

    @pl.when(sid == NS - 1)
    def _():
        pltpu.sync_copy(acc.at[pl.ds((NS - 1) * ZROW, ZLAST)],
                        out_hbm.at[cid, pl.ds((NS - 1) * ZROW, ZLAST)])


HW = CW // 2


def _make_conv(Dw):
    @functools.partial(
        pl.kernel,
        out_type=jax.ShapeDtypeStruct((NC, N, Dw), jnp.float32),
        mesh=plsc.VectorSubcoreMesh(**_MESH),
        scratch_types=[
            pltpu.VMEM((CHUNKS, CW), jnp.int32),
            pltpu.VMEM((CHUNKS, CW), jnp.int32),
            pltpu.VMEM((HW, Dw), jnp.float32),
            pltpu.VMEM((HW, Dw), jnp.float32),
            pltpu.VMEM_SHARED((AROWS, Dw), jnp.float32),
            pltpu.SemaphoreType.DMA,
            pltpu.SemaphoreType.DMA,
        ],
    )
    def k(src_hbm, dst_hbm, xw_hbm, zeros_hbm, out_hbm,
          sidx, didx, buf0, buf1, acc, sem0, sem1):
        cid = lax.axis_index("c")
        sid = lax.axis_index("s")
        w = _wid()
        pltpu.sync_copy(src_hbm.at[w], sidx)
        pltpu.sync_copy(dst_hbm.at[w], didx)
        _init_acc(sid, zeros_hbm, acc)
        plsc.subcore_barrier()

        def s_at(j, o):
            return sidx.at[j, pl.ds(o, HW)]

        def fire(j, o, buf, sem):
            pltpu.async_copy(xw_hbm.at[s_at(j, o)], buf, sem)

        def drain(j, o, buf, sem):
            pltpu.make_async_copy(xw_hbm.at[s_at(j, o)], buf, sem).wait()

        def add(j, o, buf):
            pltpu.sync_copy(buf, acc.at[didx.at[j, pl.ds(o, HW)]], add=True)

        fire(0, 0, buf0, sem0)

        def body(j, carry):
            fire(j, HW, buf1, sem1)
            drain(j, 0, buf0, sem0)
            add(j, 0, buf0)
            fire(j + 1, 0, buf0, sem0)
            drain(j, HW, buf1, sem1)
            add(j, HW, buf1)
            return carry

        lax.fori_loop(0, CHUNKS - 1, body, 0)
        j = CHUNKS - 1
        fire(j, HW, buf1, sem1)
        drain(j, 0, buf0, sem0)
        add(j, 0, buf0)
        drain(j, HW, buf1, sem1)
        add(j, HW, buf1)

        plsc.subcore_barrier()
        _read_acc(cid, sid, acc, out_hbm)

    return k


_conv128 = _make_conv(D)


def _sc_decode(la_p, lb_p, z):
    @functools.partial(
        pl.kernel,
        out_type=jax.ShapeDtypeStruct((NW, CHUNKS, CW, 16), jnp.float32),
        mesh=plsc.VectorSubcoreMesh(**_MESH),
        scratch_types=[
            pltpu.VMEM((CHUNKS, CW), jnp.int32),
            pltpu.VMEM((CHUNKS, CW), jnp.int32),
            pltpu.VMEM((CW, D), jnp.float32),
            pltpu.VMEM((CW, D), jnp.float32),
            pltpu.VMEM((CW, D), jnp.float32),
            pltpu.VMEM((CW, D), jnp.float32),
            pltpu.VMEM((CW, 16), jnp.float32),
            pltpu.SemaphoreType.DMA,
            pltpu.SemaphoreType.DMA,
        ],
    )
    def k(la_hbm, lb_hbm, z_hbm, out_hbm,
          aidx, bidx, a0, b0, a1, b1, sums, semA, semB):
        w = _wid()
        pltpu.sync_copy(la_hbm.at[w], aidx)
        pltpu.sync_copy(lb_hbm.at[w], bidx)

        def fire(j, ab, bb, sem):
            pltpu.async_copy(z_hbm.at[aidx.at[j]], ab, sem)
            pltpu.async_copy(z_hbm.at[bidx.at[j]], bb, sem)

        def drain(j, ab, bb, sem):
            pltpu.make_async_copy(z_hbm.at[aidx.at[j]], ab, sem).wait()
            pltpu.make_async_copy(z_hbm.at[bidx.at[j]], bb, sem).wait()

        def compute(j, ab, bb):
            def edge(e, carry):
                sums[e] = (ab[e, pl.ds(0, 16)] * bb[e, pl.ds(0, 16)]
                           + ab[e, pl.ds(16, 16)] * bb[e, pl.ds(16, 16)]
                           + ab[e, pl.ds(32, 16)] * bb[e, pl.ds(32, 16)]
                           + ab[e, pl.ds(48, 16)] * bb[e, pl.ds(48, 16)])
                return carry

            lax.fori_loop(0, CW, edge, 0)
            pltpu.sync_copy(sums, out_hbm.at[w, j])

        fire(0, a0, b0, semA)

        def pair(p, carry):
            j = 2 * p
            fire(j + 1, a1, b1, semB)
            drain(j, a0, b0, semA)
            compute(j, a0, b0)
            fire(j + 2, a0, b0, semA)
            drain(j + 1, a1, b1, semB)
            compute(j + 1, a1, b1)
            return carry

        lax.fori_loop(0, (CHUNKS - 1) // 2, pair, 0)
        drain(CHUNKS - 1, a0, b0, semA)
        compute(CHUNKS - 1, a0, b0)

    return k(la_p, lb_p, z)


_DEC_R = NW * CHUNKS


def _dec_reduce_body(sums_ref, out_ref):
    out_ref[...] = jnp.sum(sums_ref[...], axis=-1)


def _dec_reduce(sums):
    return pl.pallas_call(
        _dec_reduce_body,
        grid=(_DEC_R // 32,),
        in_specs=[pl.BlockSpec((32, CW, 16), lambda b: (b, 0, 0))],
        out_specs=pl.BlockSpec((32, CW), lambda b: (b, 0)),
        out_shape=jax.ShapeDtypeStruct((_DEC_R, CW), jnp.float32),
    )(sums.reshape(_DEC_R, CW, 16))


_RB = 1000


def _dinv_of(degp):
    deg = degp[0, :, 0:1] + degp[1, :, 0:1] + 1.0
    return lax.rsqrt(deg)


def _tc1_body(x_ref, wih_ref, bias_ref, w1_ref, degp_ref, xw1p_ref):
    x = x_ref[...]
    g = lax.dot_general(x, wih_ref[...], (((1,), (1,)), ((), ())),
                        preferred_element_type=jnp.float32)
    g = g + bias_ref[...]
    i = g[:, 0:D]
    gg = g[:, 2 * D:3 * D]
    o = g[:, 3 * D:4 * D]
    c = jax.nn.sigmoid(i) * jnp.tanh(gg)
    h = jax.nn.sigmoid(o) * jnp.tanh(c)
    dinv = _dinv_of(degp_ref[...])
    xw1p_ref[...] = jnp.dot(h, w1_ref[...],
                            preferred_element_type=jnp.float32) * dinv


def _tc1(x, W_ih, bias, W1, degp):
    grid = N // _RB
    return pl.pallas_call(
        _tc1_body,
        grid=(grid,),
        in_specs=[
            pl.BlockSpec((_RB, D), lambda b: (b, 0)),
            pl.BlockSpec((4 * D, D), lambda b: (0, 0)),
            pl.BlockSpec((1, 4 * D), lambda b: (0, 0)),
            pl.BlockSpec((D, H2), lambda b: (0, 0)),
            pl.BlockSpec((NC, _RB, D), lambda b: (0, b, 0)),
        ],
        out_specs=pl.BlockSpec((_RB, H2), lambda b: (b, 0)),
        out_shape=jax.ShapeDtypeStruct((N, H2), jnp.float32),
    )(x, W_ih, bias, W1, degp)


def _tc2_body(accp_ref, xw1p_ref, degp_ref, b1_ref, w2_ref, xw2p_ref):
    dinv = _dinv_of(degp_ref[...])
    s = accp_ref[0] + accp_ref[1] + xw1p_ref[...]
    h1 = jnp.maximum(dinv * s + b1_ref[...], 0.0)
    xw2p_ref[...] = jnp.dot(h1, w2_ref[...],
                            preferred_element_type=jnp.float32) * dinv


def _tc2(acc1, xw1p, degp, b1r, W2p):
    grid = N // _RB
    return pl.pallas_call(
        _tc2_body,
        grid=(grid,),
        in_specs=[
            pl.BlockSpec((NC, _RB, H2), lambda b: (0, b, 0)),
            pl.BlockSpec((_RB, H2), lambda b: (b, 0)),
            pl.BlockSpec((NC, _RB, D), lambda b: (0, b, 0)),
            pl.BlockSpec((1, H2), lambda b: (0, 0)),
            pl.BlockSpec((H2, D), lambda b: (0, 0)),
        ],
        out_specs=pl.BlockSpec((_RB, D), lambda b: (b, 0)),
        out_shape=jax.ShapeDtypeStruct((N, D), jnp.float32),
    )(acc1, xw1p, degp, b1r, W2p)


def _tc3_body(accp_ref, xw2p_ref, degp_ref, b2_ref, z_ref):
    dinv = _dinv_of(degp_ref[...])
    z_ref[...] = dinv * (accp_ref[0] + accp_ref[1] + xw2p_ref[...]) \
        + b2_ref[...]


def _tc3(acc2, xw2p, degp, b2r):
    grid = N // _RB
    return pl.pallas_call(
        _tc3_body,
        grid=(grid,),
        in_specs=[
            pl.BlockSpec((NC, _RB, D), lambda b: (0, b, 0)),
            pl.BlockSpec((_RB, D), lambda b: (b, 0)),
            pl.BlockSpec((NC, _RB, D), lambda b: (0, b, 0)),
            pl.BlockSpec((1, D), lambda b: (0, 0)),
        ],
        out_specs=pl.BlockSpec((_RB, D), lambda b: (b, 0)),
        out_shape=jax.ShapeDtypeStruct((N, D), jnp.float32),
    )(acc2, xw2p, degp, b2r)


def kernel(x, edge_index, edge_label_index, W_ih, W_hh, b_ih, b_hh,
           W1, b1, W2, b2):
    pad = EPAD - E
    zpad = jnp.zeros((pad,), jnp.int32)
    src_p = jnp.concatenate([edge_index[0], zpad]).reshape(NW, CHUNKS, CW)
    dst_p = jnp.concatenate(
        [edge_index[1], jnp.full((pad,), N, jnp.int32)]
    ).reshape(NW, CHUNKS, CW)
    la_p = jnp.concatenate([edge_label_index[0], zpad]).reshape(NW, CHUNKS, CW)
    lb_p = jnp.concatenate([edge_label_index[1], zpad]).reshape(NW, CHUNKS, CW)

    zeros128 = jnp.zeros((N, D), jnp.float32)
    bias = (b_ih + b_hh).reshape(1, 4 * D)
    b1r = b1.reshape(1, H2)
    W2p = jnp.pad(W2, ((0, 0), (0, D - C)))
    b2r = jnp.pad(b2, (0, D - C)).reshape(1, D)

    onesN = jnp.ones((N, D), jnp.float32)
    degp = _conv128(dst_p, dst_p, onesN, zeros128)
    xw1p = _tc1(x, W_ih, bias, W1, degp)
    acc1 = _conv128(src_p, dst_p, xw1p, zeros128)
    xw2p = _tc2(acc1, xw1p, degp, b1r, W2p)
    acc2 = _conv128(src_p, dst_p, xw2p, zeros128)
    z = _tc3(acc2, xw2p, degp, b2r)
    outp = _dec_reduce(_sc_decode(la_p, lb_p, z))
    return outp.reshape(-1)[:E]


# speedup vs baseline: 7.7972x; 1.1002x over previous
import functools

import jax
import jax.numpy as jnp
from jax import lax
from jax.experimental import pallas as pl
from jax.experimental.pallas import tpu as pltpu
from jax.experimental.pallas import tpu_sc as plsc

N = 10000
E = 320000
D = 128
H2 = 128
C = 64

NC, NS = 2, 16
NW = NC * NS
CW = 128
CHUNKS = -(-E // (NW * CW))
EPW = CHUNKS * CW
EPAD = EPW * NW
ZROW = 632
ZLAST = N - ZROW * (NS - 1)
AROWS = N + 8

_MESH = dict(core_axis_name="c", subcore_axis_name="s", num_cores=NC,
             num_subcores=NS)


def _wid():
    return lax.axis_index("c") * NS + lax.axis_index("s")


def _init_acc(sid, zeros_hbm, acc):
    @pl.when(sid < NS - 1)
    def _():
        pltpu.sync_copy(zeros_hbm.at[pl.ds(sid * ZROW, ZROW)],
                        acc.at[pl.ds(sid * ZROW, ZROW)])

    @pl.when(sid == NS - 1)
    def _():
        pltpu.sync_copy(zeros_hbm.at[pl.ds((NS - 1) * ZROW, ZLAST)],
                        acc.at[pl.ds((NS - 1) * ZROW, ZLAST)])


def _read_acc(cid, sid, acc, out_hbm):
    @pl.when(sid < NS - 1)
    def _():
        pltpu.sync_copy(acc.at[pl.ds(sid * ZROW, ZROW)],
                        out_hbm.at[cid, pl.ds(sid * ZROW, ZROW)])

    @pl.when(sid == NS - 1)
    def _():
        pltpu.sync_copy(acc.at[pl.ds((NS - 1) * ZROW, ZLAST)],
                        out_hbm.at[cid, pl.ds((NS - 1) * ZROW, ZLAST)])


HW = CW // 2


def _make_conv(Dw):
    @functools.partial(
        pl.kernel,
        out_type=jax.ShapeDtypeStruct((NC, N, Dw), jnp.float32),
        mesh=plsc.VectorSubcoreMesh(**_MESH),
        scratch_types=[
            pltpu.VMEM((CHUNKS, CW), jnp.int32),
            pltpu.VMEM((CHUNKS, CW), jnp.int32),
            pltpu.VMEM((HW, Dw), jnp.float32),
            pltpu.VMEM((HW, Dw), jnp.float32),
            pltpu.VMEM_SHARED((AROWS, Dw), jnp.float32),
            pltpu.SemaphoreType.DMA,
            pltpu.SemaphoreType.DMA,
        ],
    )
    def k(src_hbm, dst_hbm, xw_hbm, zeros_hbm, out_hbm,
          sidx, didx, buf0, buf1, acc, sem0, sem1):
        cid = lax.axis_index("c")
        sid = lax.axis_index("s")
        w = _wid()
        pltpu.sync_copy(src_hbm.at[w], sidx)
        pltpu.sync_copy(dst_hbm.at[w], didx)
        _init_acc(sid, zeros_hbm, acc)
        plsc.subcore_barrier()

        def s_at(j, o):
            return sidx.at[j, pl.ds(o, HW)]

        def fire(j, o, buf, sem):
            pltpu.async_copy(xw_hbm.at[s_at(j, o)], buf, sem)

        def drain(j, o, buf, sem):
            pltpu.make_async_copy(xw_hbm.at[s_at(j, o)], buf, sem).wait()

        def add(j, o, buf):
            pltpu.sync_copy(buf, acc.at[didx.at[j, pl.ds(o, HW)]], add=True)

        fire(0, 0, buf0, sem0)

        def body(j, carry):
            fire(j, HW, buf1, sem1)
            drain(j, 0, buf0, sem0)
            add(j, 0, buf0)
            fire(j + 1, 0, buf0, sem0)
            drain(j, HW, buf1, sem1)
            add(j, HW, buf1)
            return carry

        lax.fori_loop(0, CHUNKS - 1, body, 0)
        j = CHUNKS - 1
        fire(j, HW, buf1, sem1)
        drain(j, 0, buf0, sem0)
        add(j, 0, buf0)
        drain(j, HW, buf1, sem1)
        add(j, HW, buf1)

        plsc.subcore_barrier()
        _read_acc(cid, sid, acc, out_hbm)

    return k


_conv128 = _make_conv(D)


def _sc_degree(dst_p, ones64, zeros128):
    @functools.partial(
        pl.kernel,
        out_type=jax.ShapeDtypeStruct((NC, N, D), jnp.float32),
        mesh=plsc.VectorSubcoreMesh(**_MESH),
        scratch_types=[
            pltpu.VMEM((CHUNKS, CW), jnp.int32),
            pltpu.VMEM((HW, D), jnp.float32),
            pltpu.VMEM_SHARED((AROWS, D), jnp.float32),
        ],
    )
    def k(dst_hbm, ones_hbm, zeros_hbm, out_hbm, didx, buf, acc):
        cid = lax.axis_index("c")
        sid = lax.axis_index("s")
        w = _wid()
        pltpu.sync_copy(dst_hbm.at[w], didx)
        pltpu.sync_copy(ones_hbm, buf)
        _init_acc(sid, zeros_hbm, acc)
        plsc.subcore_barrier()

        def chunk(j, carry):
            pltpu.sync_copy(buf, acc.at[didx.at[j, pl.ds(0, HW)]], add=True)
            pltpu.sync_copy(buf, acc.at[didx.at[j, pl.ds(HW, HW)]], add=True)
            return carry

        lax.fori_loop(0, CHUNKS, chunk, 0)
        plsc.subcore_barrier()
        _read_acc(cid, sid, acc, out_hbm)

    return k(dst_p, ones64, zeros128)


def _sc_decode(la_p, lb_p, z):
    @functools.partial(
        pl.kernel,
        out_type=jax.ShapeDtypeStruct((NW, CHUNKS, CW, 16), jnp.float32),
        mesh=plsc.VectorSubcoreMesh(**_MESH),
        scratch_types=[
            pltpu.VMEM((CHUNKS, CW), jnp.int32),
            pltpu.VMEM((CHUNKS, CW), jnp.int32),
            pltpu.VMEM((CW, D), jnp.float32),
            pltpu.VMEM((CW, D), jnp.float32),
            pltpu.VMEM((CW, D), jnp.float32),
            pltpu.VMEM((CW, D), jnp.float32),
            pltpu.VMEM((CW, 16), jnp.float32),
            pltpu.SemaphoreType.DMA,
            pltpu.SemaphoreType.DMA,
        ],
    )
    def k(la_hbm, lb_hbm, z_hbm, out_hbm,
          aidx, bidx, a0, b0, a1, b1, sums, semA, semB):
        w = _wid()
        pltpu.sync_copy(la_hbm.at[w], aidx)
        pltpu.sync_copy(lb_hbm.at[w], bidx)

        def fire(j, ab, bb, sem):
            pltpu.async_copy(z_hbm.at[aidx.at[j]], ab, sem)
            pltpu.async_copy(z_hbm.at[bidx.at[j]], bb, sem)

        def drain(j, ab, bb, sem):
            pltpu.make_async_copy(z_hbm.at[aidx.at[j]], ab, sem).wait()
            pltpu.make_async_copy(z_hbm.at[bidx.at[j]], bb, sem).wait()

        def compute(j, ab, bb):
            def edge(e, carry):
                sums[e] = (ab[e, pl.ds(0, 16)] * bb[e, pl.ds(0, 16)]
                           + ab[e, pl.ds(16, 16)] * bb[e, pl.ds(16, 16)]
                           + ab[e, pl.ds(32, 16)] * bb[e, pl.ds(32, 16)]
                           + ab[e, pl.ds(48, 16)] * bb[e, pl.ds(48, 16)])
                return carry

            lax.fori_loop(0, CW, edge, 0, unroll=8)
            pltpu.sync_copy(sums, out_hbm.at[w, j])

        fire(0, a0, b0, semA)

        def pair(p, carry):
            j = 2 * p
            fire(j + 1, a1, b1, semB)
            drain(j, a0, b0, semA)
            compute(j, a0, b0)
            fire(j + 2, a0, b0, semA)
            drain(j + 1, a1, b1, semB)
            compute(j + 1, a1, b1)
            return carry

        lax.fori_loop(0, (CHUNKS - 1) // 2, pair, 0)
        drain(CHUNKS - 1, a0, b0, semA)
        compute(CHUNKS - 1, a0, b0)

    return k(la_p, lb_p, z)


_DEC_R = NW * CHUNKS


def _dec_reduce_body(sums_ref, out_ref):
    out_ref[...] = jnp.sum(sums_ref[...], axis=-1)


def _dec_reduce(sums):
    return pl.pallas_call(
        _dec_reduce_body,
        grid=(_DEC_R // 32,),
        in_specs=[pl.BlockSpec((32, CW, 16), lambda b: (b, 0, 0))],
        out_specs=pl.BlockSpec((32, CW), lambda b: (b, 0)),
        out_shape=jax.ShapeDtypeStruct((_DEC_R, CW), jnp.float32),
    )(sums.reshape(_DEC_R, CW, 16))


_RB = 1000


def _dinv_of(degp):
    deg = degp[0, :, 0:1] + degp[1, :, 0:1] + 1.0
    return lax.rsqrt(deg)


def _tc1_body(x_ref, wih_ref, bias_ref, w1_ref, degp_ref, xw1p_ref):
    x = x_ref[...]
    g = lax.dot_general(x, wih_ref[...], (((1,), (1,)), ((), ())),
                        preferred_element_type=jnp.float32)
    g = g + bias_ref[...]
    i = g[:, 0:D]
    gg = g[:, 2 * D:3 * D]
    o = g[:, 3 * D:4 * D]
    c = jax.nn.sigmoid(i) * jnp.tanh(gg)
    h = jax.nn.sigmoid(o) * jnp.tanh(c)
    dinv = _dinv_of(degp_ref[...])
    xw1p_ref[...] = jnp.dot(h, w1_ref[...],
                            preferred_element_type=jnp.float32) * dinv


def _tc1(x, W_ih, bias, W1, degp):
    grid = N // _RB
    return pl.pallas_call(
        _tc1_body,
        grid=(grid,),
        in_specs=[
            pl.BlockSpec((_RB, D), lambda b: (b, 0)),
            pl.BlockSpec((4 * D, D), lambda b: (0, 0)),
            pl.BlockSpec((1, 4 * D), lambda b: (0, 0)),
            pl.BlockSpec((D, H2), lambda b: (0, 0)),
            pl.BlockSpec((NC, _RB, D), lambda b: (0, b, 0)),
        ],
        out_specs=pl.BlockSpec((_RB, H2), lambda b: (b, 0)),
        out_shape=jax.ShapeDtypeStruct((N, H2), jnp.float32),
    )(x, W_ih, bias, W1, degp)


def _tc2_body(accp_ref, xw1p_ref, degp_ref, b1_ref, w2_ref, xw2p_ref):
    dinv = _dinv_of(degp_ref[...])
    s = accp_ref[0] + accp_ref[1] + xw1p_ref[...]
    h1 = jnp.maximum(dinv * s + b1_ref[...], 0.0)
    xw2p_ref[...] = jnp.dot(h1, w2_ref[...],
                            preferred_element_type=jnp.float32) * dinv


def _tc2(acc1, xw1p, degp, b1r, W2p):
    grid = N // _RB
    return pl.pallas_call(
        _tc2_body,
        grid=(grid,),
        in_specs=[
            pl.BlockSpec((NC, _RB, H2), lambda b: (0, b, 0)),
            pl.BlockSpec((_RB, H2), lambda b: (b, 0)),
            pl.BlockSpec((NC, _RB, D), lambda b: (0, b, 0)),
            pl.BlockSpec((1, H2), lambda b: (0, 0)),
            pl.BlockSpec((H2, D), lambda b: (0, 0)),
        ],
        out_specs=pl.BlockSpec((_RB, D), lambda b: (b, 0)),
        out_shape=jax.ShapeDtypeStruct((N, D), jnp.float32),
    )(acc1, xw1p, degp, b1r, W2p)


def _tc3_body(accp_ref, xw2p_ref, degp_ref, b2_ref, z_ref):
    dinv = _dinv_of(degp_ref[...])
    z_ref[...] = dinv * (accp_ref[0] + accp_ref[1] + xw2p_ref[...]) \
        + b2_ref[...]


def _tc3(acc2, xw2p, degp, b2r):
    grid = N // _RB
    return pl.pallas_call(
        _tc3_body,
        grid=(grid,),
        in_specs=[
            pl.BlockSpec((NC, _RB, D), lambda b: (0, b, 0)),
            pl.BlockSpec((_RB, D), lambda b: (b, 0)),
            pl.BlockSpec((NC, _RB, D), lambda b: (0, b, 0)),
            pl.BlockSpec((1, D), lambda b: (0, 0)),
        ],
        out_specs=pl.BlockSpec((_RB, D), lambda b: (b, 0)),
        out_shape=jax.ShapeDtypeStruct((N, D), jnp.float32),
    )(acc2, xw2p, degp, b2r)


def kernel(x, edge_index, edge_label_index, W_ih, W_hh, b_ih, b_hh,
           W1, b1, W2, b2):
    pad = EPAD - E
    zpad = jnp.zeros((pad,), jnp.int32)
    src_p = jnp.concatenate([edge_index[0], zpad]).reshape(NW, CHUNKS, CW)
    dst_p = jnp.concatenate(
        [edge_index[1], jnp.full((pad,), N, jnp.int32)]
    ).reshape(NW, CHUNKS, CW)
    la_p = jnp.concatenate([edge_label_index[0], zpad]).reshape(NW, CHUNKS, CW)
    lb_p = jnp.concatenate([edge_label_index[1], zpad]).reshape(NW, CHUNKS, CW)

    zeros128 = jnp.zeros((N, D), jnp.float32)
    bias = (b_ih + b_hh).reshape(1, 4 * D)
    b1r = b1.reshape(1, H2)
    W2p = jnp.pad(W2, ((0, 0), (0, D - C)))
    b2r = jnp.pad(b2, (0, D - C)).reshape(1, D)

    ones64 = jnp.ones((HW, D), jnp.float32)
    degp = _sc_degree(dst_p, ones64, zeros128)
    xw1p = _tc1(x, W_ih, bias, W1, degp)
    acc1 = _conv128(src_p, dst_p, xw1p, zeros128)
    xw2p = _tc2(acc1, xw1p, degp, b1r, W2p)
    acc2 = _conv128(src_p, dst_p, xw2p, zeros128)
    z = _tc3(acc2, xw2p, degp, b2r)
    outp = _dec_reduce(_sc_decode(la_p, lb_p, z))
    return outp.reshape(-1)[:E]


# speedup vs baseline: 10.1591x; 1.3029x over previous
import functools

import jax
import jax.numpy as jnp
from jax import lax
from jax.experimental import pallas as pl
from jax.experimental.pallas import tpu as pltpu
from jax.experimental.pallas import tpu_sc as plsc

N = 10000
E = 320000
D = 128
H2 = 128
C = 64

NC, NS = 2, 16
NW = NC * NS
CW = 128
CHUNKS = -(-E // (NW * CW))
EPW = CHUNKS * CW
EPAD = EPW * NW
ZROW = 632
ZLAST = N - ZROW * (NS - 1)
AROWS = N + 8

_MESH = dict(core_axis_name="c", subcore_axis_name="s", num_cores=NC,
             num_subcores=NS)


def _wid():
    return lax.axis_index("c") * NS + lax.axis_index("s")


def _init_acc(sid, zeros_hbm, acc):
    @pl.when(sid < NS - 1)
    def _():
        pltpu.sync_copy(zeros_hbm.at[pl.ds(sid * ZROW, ZROW)],
                        acc.at[pl.ds(sid * ZROW, ZROW)])

    @pl.when(sid == NS - 1)
    def _():
        pltpu.sync_copy(zeros_hbm.at[pl.ds((NS - 1) * ZROW, ZLAST)],
                        acc.at[pl.ds((NS - 1) * ZROW, ZLAST)])


def _read_acc(cid, sid, acc, out_hbm):
    @pl.when(sid < NS - 1)
    def _():
        pltpu.sync_copy(acc.at[pl.ds(sid * ZROW, ZROW)],
                        out_hbm.at[cid, pl.ds(sid * ZROW, ZROW)])

    @pl.when(sid == NS - 1)
    def _():
        pltpu.sync_copy(acc.at[pl.ds((NS - 1) * ZROW, ZLAST)],
                        out_hbm.at[cid, pl.ds((NS - 1) * ZROW, ZLAST)])


HW = CW // 2


def _make_conv(Dw):
    @functools.partial(
        pl.kernel,
        out_type=jax.ShapeDtypeStruct((NC, N, Dw), jnp.float32),
        mesh=plsc.VectorSubcoreMesh(**_MESH),
        scratch_types=[
            pltpu.VMEM((CHUNKS, CW), jnp.int32),
            pltpu.VMEM((CHUNKS, CW), jnp.int32),
            pltpu.VMEM((HW, Dw), jnp.float32),
            pltpu.VMEM((HW, Dw), jnp.float32),
            pltpu.VMEM_SHARED((AROWS, Dw), jnp.float32),
            pltpu.SemaphoreType.DMA,
            pltpu.SemaphoreType.DMA,
        ],
    )
    def k(src_hbm, dst_hbm, xw_hbm, zeros_hbm, out_hbm,
          sidx, didx, buf0, buf1, acc, sem0, sem1):
        cid = lax.axis_index("c")
        sid = lax.axis_index("s")
        w = _wid()
        pltpu.sync_copy(src_hbm.at[w], sidx)
        pltpu.sync_copy(dst_hbm.at[w], didx)
        _init_acc(sid, zeros_hbm, acc)
        plsc.subcore_barrier()

        def s_at(j, o):
            return sidx.at[j, pl.ds(o, HW)]

        def fire(j, o, buf, sem):
            pltpu.async_copy(xw_hbm.at[s_at(j, o)], buf, sem)

        def drain(j, o, buf, sem):
            pltpu.make_async_copy(xw_hbm.at[s_at(j, o)], buf, sem).wait()

        def add(j, o, buf):
            pltpu.sync_copy(buf, acc.at[didx.at[j, pl.ds(o, HW)]], add=True)

        fire(0, 0, buf0, sem0)

        def body(j, carry):
            fire(j, HW, buf1, sem1)
            drain(j, 0, buf0, sem0)
            add(j, 0, buf0)
            fire(j + 1, 0, buf0, sem0)
            drain(j, HW, buf1, sem1)
            add(j, HW, buf1)
            return carry

        lax.fori_loop(0, CHUNKS - 1, body, 0)
        j = CHUNKS - 1
        fire(j, HW, buf1, sem1)
        drain(j, 0, buf0, sem0)
        add(j, 0, buf0)
        drain(j, HW, buf1, sem1)
        add(j, HW, buf1)

        plsc.subcore_barrier()
        _read_acc(cid, sid, acc, out_hbm)

    return k


_conv128 = _make_conv(D)


def _sc_degree(dst_p, ones64, zeros128):
    @functools.partial(
        pl.kernel,
        out_type=jax.ShapeDtypeStruct((NC, N, D), jnp.float32),
        mesh=plsc.VectorSubcoreMesh(**_MESH),
        scratch_types=[
            pltpu.VMEM((CHUNKS, CW), jnp.int32),
            pltpu.VMEM((HW, D), jnp.float32),
            pltpu.VMEM_SHARED((AROWS, D), jnp.float32),
        ],
    )
    def k(dst_hbm, ones_hbm, zeros_hbm, out_hbm, didx, buf, acc):
        cid = lax.axis_index("c")
        sid = lax.axis_index("s")
        w = _wid()
        pltpu.sync_copy(dst_hbm.at[w], didx)
        pltpu.sync_copy(ones_hbm, buf)
        _init_acc(sid, zeros_hbm, acc)
        plsc.subcore_barrier()

        def chunk(j, carry):
            pltpu.sync_copy(buf, acc.at[didx.at[j, pl.ds(0, HW)]], add=True)
            pltpu.sync_copy(buf, acc.at[didx.at[j, pl.ds(HW, HW)]], add=True)
            return carry

        lax.fori_loop(0, CHUNKS, chunk, 0)
        plsc.subcore_barrier()
        _read_acc(cid, sid, acc, out_hbm)

    return k(dst_p, ones64, zeros128)


def _sc_decode(la_p, lb_p, z):
    @functools.partial(
        pl.kernel,
        out_type=jax.ShapeDtypeStruct((NW, CHUNKS, CW, 16), jnp.float32),
        mesh=plsc.VectorSubcoreMesh(**_MESH),
        scratch_types=[
            pltpu.VMEM((CHUNKS, CW), jnp.int32),
            pltpu.VMEM((CHUNKS, CW), jnp.int32),
            pltpu.VMEM_SHARED((N, D), jnp.float32),
            pltpu.VMEM((HW, D), jnp.float32),
            pltpu.VMEM((HW, D), jnp.float32),
            pltpu.VMEM((HW, 16), jnp.float32),
        ],
    )
    def k(la_hbm, lb_hbm, z_hbm, out_hbm, aidx, bidx, zs, ab, bb, sums):
        sid = lax.axis_index("s")
        w = _wid()
        pltpu.sync_copy(la_hbm.at[w], aidx)
        pltpu.sync_copy(lb_hbm.at[w], bidx)

        @pl.when(sid < NS - 1)
        def _():
            pltpu.sync_copy(z_hbm.at[pl.ds(sid * ZROW, ZROW)],
                            zs.at[pl.ds(sid * ZROW, ZROW)])

        @pl.when(sid == NS - 1)
        def _():
            pltpu.sync_copy(z_hbm.at[pl.ds((NS - 1) * ZROW, ZLAST)],
                            zs.at[pl.ds((NS - 1) * ZROW, ZLAST)])

        plsc.subcore_barrier()

        def half(j, o):
            pltpu.sync_copy(zs.at[aidx.at[j, pl.ds(o, HW)]], ab)
            pltpu.sync_copy(zs.at[bidx.at[j, pl.ds(o, HW)]], bb)
            def edge(e, carry):
                def seg(c):
                    return ab[e, pl.ds(c, 16)] * bb[e, pl.ds(c, 16)]

                sums[e] = seg(0) + seg(16) + seg(32) + seg(48)
                return carry

            lax.fori_loop(0, HW, edge, 0, unroll=8)
            pltpu.sync_copy(sums, out_hbm.at[w, j, pl.ds(o, HW)])

        def chunk(j, carry):
            half(j, 0)
            half(j, HW)
            return carry

        lax.fori_loop(0, CHUNKS, chunk, 0)

    return k(la_p, lb_p, z)


_DEC_R = NW * CHUNKS


def _dec_reduce_body(sums_ref, out_ref):
    out_ref[...] = jnp.sum(sums_ref[...], axis=-1)


def _dec_reduce(sums):
    return pl.pallas_call(
        _dec_reduce_body,
        grid=(_DEC_R // 32,),
        in_specs=[pl.BlockSpec((32, CW, 16), lambda b: (b, 0, 0))],
        out_specs=pl.BlockSpec((32, CW), lambda b: (b, 0)),
        out_shape=jax.ShapeDtypeStruct((_DEC_R, CW), jnp.float32),
    )(sums.reshape(_DEC_R, CW, 16))


_RB = 1000


def _dinv_of(degp):
    deg = degp[0, :, 0:1] + degp[1, :, 0:1] + 1.0
    return lax.rsqrt(deg)


def _tc1_body(x_ref, wih_ref, bias_ref, w1_ref, degp_ref, xw1p_ref):
    x = x_ref[...]
    g = lax.dot_general(x, wih_ref[...], (((1,), (1,)), ((), ())),
                        preferred_element_type=jnp.float32)
    g = g + bias_ref[...]
    i = g[:, 0:D]
    gg = g[:, 2 * D:3 * D]
    o = g[:, 3 * D:4 * D]
    c = jax.nn.sigmoid(i) * jnp.tanh(gg)
    h = jax.nn.sigmoid(o) * jnp.tanh(c)
    dinv = _dinv_of(degp_ref[...])
    xw1p_ref[...] = jnp.dot(h, w1_ref[...],
                            preferred_element_type=jnp.float32) * dinv


def _tc1(x, W_ih, bias, W1, degp):
    grid = N // _RB
    return pl.pallas_call(
        _tc1_body,
        grid=(grid,),
        in_specs=[
            pl.BlockSpec((_RB, D), lambda b: (b, 0)),
            pl.BlockSpec((4 * D, D), lambda b: (0, 0)),
            pl.BlockSpec((1, 4 * D), lambda b: (0, 0)),
            pl.BlockSpec((D, H2), lambda b: (0, 0)),
            pl.BlockSpec((NC, _RB, D), lambda b: (0, b, 0)),
        ],
        out_specs=pl.BlockSpec((_RB, H2), lambda b: (b, 0)),
        out_shape=jax.ShapeDtypeStruct((N, H2), jnp.float32),
    )(x, W_ih, bias, W1, degp)


def _tc2_body(accp_ref, xw1p_ref, degp_ref, b1_ref, w2_ref, xw2p_ref):
    dinv = _dinv_of(degp_ref[...])
    s = accp_ref[0] + accp_ref[1] + xw1p_ref[...]
    h1 = jnp.maximum(dinv * s + b1_ref[...], 0.0)
    xw2p_ref[...] = jnp.dot(h1, w2_ref[...],
                            preferred_element_type=jnp.float32) * dinv


def _tc2(acc1, xw1p, degp, b1r, W2p):
    grid = N // _RB
    return pl.pallas_call(
        _tc2_body,
        grid=(grid,),
        in_specs=[
            pl.BlockSpec((NC, _RB, H2), lambda b: (0, b, 0)),
            pl.BlockSpec((_RB, H2), lambda b: (b, 0)),
            pl.BlockSpec((NC, _RB, D), lambda b: (0, b, 0)),
            pl.BlockSpec((1, H2), lambda b: (0, 0)),
            pl.BlockSpec((H2, D), lambda b: (0, 0)),
        ],
        out_specs=pl.BlockSpec((_RB, D), lambda b: (b, 0)),
        out_shape=jax.ShapeDtypeStruct((N, D), jnp.float32),
    )(acc1, xw1p, degp, b1r, W2p)


def _tc3_body(accp_ref, xw2p_ref, degp_ref, b2_ref, z_ref):
    dinv = _dinv_of(degp_ref[...])
    z_ref[...] = dinv * (accp_ref[0] + accp_ref[1] + xw2p_ref[...]) \
        + b2_ref[...]


def _tc3(acc2, xw2p, degp, b2r):
    grid = N // _RB
    return pl.pallas_call(
        _tc3_body,
        grid=(grid,),
        in_specs=[
            pl.BlockSpec((NC, _RB, D), lambda b: (0, b, 0)),
            pl.BlockSpec((_RB, D), lambda b: (b, 0)),
            pl.BlockSpec((NC, _RB, D), lambda b: (0, b, 0)),
            pl.BlockSpec((1, D), lambda b: (0, 0)),
        ],
        out_specs=pl.BlockSpec((_RB, D), lambda b: (b, 0)),
        out_shape=jax.ShapeDtypeStruct((N, D), jnp.float32),
    )(acc2, xw2p, degp, b2r)


def kernel(x, edge_index, edge_label_index, W_ih, W_hh, b_ih, b_hh,
           W1, b1, W2, b2):
    pad = EPAD - E
    zpad = jnp.zeros((pad,), jnp.int32)
    src_p = jnp.concatenate([edge_index[0], zpad]).reshape(NW, CHUNKS, CW)
    dst_p = jnp.concatenate(
        [edge_index[1], jnp.full((pad,), N, jnp.int32)]
    ).reshape(NW, CHUNKS, CW)
    la_p = jnp.concatenate([edge_label_index[0], zpad]).reshape(NW, CHUNKS, CW)
    lb_p = jnp.concatenate([edge_label_index[1], zpad]).reshape(NW, CHUNKS, CW)

    zeros128 = jnp.zeros((N, D), jnp.float32)
    bias = (b_ih + b_hh).reshape(1, 4 * D)
    b1r = b1.reshape(1, H2)
    W2p = jnp.pad(W2, ((0, 0), (0, D - C)))
    b2r = jnp.pad(b2, (0, D - C)).reshape(1, D)

    ones64 = jnp.ones((HW, D), jnp.float32)
    degp = _sc_degree(dst_p, ones64, zeros128)
    xw1p = _tc1(x, W_ih, bias, W1, degp)
    acc1 = _conv128(src_p, dst_p, xw1p, zeros128)
    xw2p = _tc2(acc1, xw1p, degp, b1r, W2p)
    acc2 = _conv128(src_p, dst_p, xw2p, zeros128)
    z = _tc3(acc2, xw2p, degp, b2r)
    outp = _dec_reduce(_sc_decode(la_p, lb_p, z))
    return outp.reshape(-1)[:E]
